# baseline (device time: 219648 ns/iter reference)
import functools

import jax
import jax.numpy as jnp
from jax import lax
from jax.experimental import pallas as pl
from jax.experimental.pallas import tpu as pltpu

N_DEV = 8
N_HOP = 3


def kernel(x, router_W, route_idx, expert_W):
    T, D = x.shape
    E_LOCAL, _, H = expert_W.shape
    N_EXP = router_W.shape[1]

    def body(x_ref, rw_ref, idx_ref, ew_ref, out_ref,
             commR_ref, commL_ref, zbuf_ref,
             sendR_sems, recvR_sems, sendL_sems, recvL_sems,
             zsend_sem, zrecv_sem, creditR_sem, creditL_sem):
        my = lax.axis_index("i")
        left = lax.rem(my - 1 + N_DEV, N_DEV)
        right = lax.rem(my + 1, N_DEV)
        zp = lax.rem(my + 4, N_DEV)

        barrier = pltpu.get_barrier_semaphore()
        for nbr in (left, right, zp):
            pl.semaphore_signal(barrier, inc=1, device_id=(nbr,),
                                device_id_type=pl.DeviceIdType.MESH)
        pl.semaphore_wait(barrier, 3)

        xv = x_ref[...]
        scores = jnp.dot(xv, rw_ref[...], preferred_element_type=jnp.float32)
        e0 = idx_ref[:, 0:1]
        e1 = idx_ref[:, 1:2]
        eids = lax.broadcasted_iota(jnp.int32, (T, N_EXP), 1)
        s0 = jnp.sum(jnp.where(eids == e0, scores, 0.0), axis=1)
        s1 = jnp.sum(jnp.where(eids == e1, scores, 0.0), axis=1)
        w0 = jax.nn.sigmoid(s0 - s1)
        w1 = 1.0 - w0
        xv16 = xv.astype(jnp.bfloat16)

        def contribution(block, origin):
            ge = (origin * E_LOCAL
                  + lax.broadcasted_iota(jnp.int32, (1, E_LOCAL), 1))
            coeff = (w0[:, None] * (e0 == ge).astype(jnp.float32)
                     + w1[:, None] * (e1 == ge).astype(jnp.float32)
                     ).astype(jnp.bfloat16)
            xs = (xv16[:, None, :] * coeff[:, :, None]).reshape(
                T, E_LOCAL * D)
            blk = block.reshape(E_LOCAL * D, H).astype(jnp.bfloat16)
            return jnp.dot(xs, blk, preferred_element_type=jnp.float32)

        blk16 = ew_ref[...].astype(jnp.bfloat16)
        commR_ref[0, ...] = blk16
        commL_ref[0, ...] = blk16

        zrdma = pltpu.make_async_remote_copy(
            src_ref=commR_ref.at[0],
            dst_ref=zbuf_ref,
            send_sem=zsend_sem,
            recv_sem=zrecv_sem,
            device_id=(zp,),
            device_id_type=pl.DeviceIdType.MESH,
        )
        zrdma.start()

        for h in range(1, N_HOP + 1):
            send_slot = (h - 1) % 2
            recv_slot = h % 2
            if h >= 2:
                pl.semaphore_wait(creditR_sem, 1)
                pl.semaphore_wait(creditL_sem, 1)
            rdmaR = pltpu.make_async_remote_copy(
                src_ref=commR_ref.at[send_slot],
                dst_ref=commR_ref.at[recv_slot],
                send_sem=sendR_sems.at[send_slot],
                recv_sem=recvR_sems.at[recv_slot],
                device_id=(right,),
                device_id_type=pl.DeviceIdType.MESH,
            )
            rdmaL = pltpu.make_async_remote_copy(
                src_ref=commL_ref.at[send_slot],
                dst_ref=commL_ref.at[recv_slot],
                send_sem=sendL_sems.at[send_slot],
                recv_sem=recvL_sems.at[recv_slot],
                device_id=(left,),
                device_id_type=pl.DeviceIdType.MESH,
            )
            rdmaR.start()
            rdmaL.start()
            if h == 1:
                out_ref[...] = contribution(ew_ref[...], my)
            else:
                out_ref[...] += contribution(
                    commR_ref[send_slot], lax.rem(my - (h - 1) + N_DEV, N_DEV))
                out_ref[...] += contribution(
                    commL_ref[send_slot], lax.rem(my + (h - 1), N_DEV))
            if h == N_HOP:
                zrdma.wait_recv()
                out_ref[...] += contribution(zbuf_ref[...], zp)
            rdmaR.wait()
            rdmaL.wait()
            if h == 1:
                zrdma.wait_send()
            if h <= N_HOP - 1:
                pl.semaphore_signal(creditR_sem, inc=1, device_id=(left,),
                                    device_id_type=pl.DeviceIdType.MESH)
                pl.semaphore_signal(creditL_sem, inc=1, device_id=(right,),
                                    device_id_type=pl.DeviceIdType.MESH)
        last = N_HOP % 2
        out_ref[...] += contribution(commR_ref[last],
                                     lax.rem(my - N_HOP + N_DEV, N_DEV))
        out_ref[...] += contribution(commL_ref[last],
                                     lax.rem(my + N_HOP, N_DEV))

        @functools.partial(pl.run_scoped,
                           exit_sem=pltpu.SemaphoreType.REGULAR)
        def _(exit_sem):
            for nbr in (left, right, zp):
                pl.semaphore_signal(exit_sem, inc=1, device_id=(nbr,),
                                    device_id_type=pl.DeviceIdType.MESH)
            pl.semaphore_wait(exit_sem, 3)

    return pl.pallas_call(
        body,
        out_shape=jax.ShapeDtypeStruct((T, H), jnp.float32),
        in_specs=[pl.BlockSpec(memory_space=pltpu.VMEM)] * 4,
        out_specs=pl.BlockSpec(memory_space=pltpu.VMEM),
        scratch_shapes=[
            pltpu.VMEM((2, E_LOCAL, D, H), jnp.bfloat16),
            pltpu.VMEM((2, E_LOCAL, D, H), jnp.bfloat16),
            pltpu.VMEM((E_LOCAL, D, H), jnp.bfloat16),
            pltpu.SemaphoreType.DMA((2,)),
            pltpu.SemaphoreType.DMA((2,)),
            pltpu.SemaphoreType.DMA((2,)),
            pltpu.SemaphoreType.DMA((2,)),
            pltpu.SemaphoreType.DMA,
            pltpu.SemaphoreType.DMA,
            pltpu.SemaphoreType.REGULAR,
            pltpu.SemaphoreType.REGULAR,
        ],
        compiler_params=pltpu.CompilerParams(
            collective_id=0,
            vmem_limit_bytes=100 * 1024 * 1024,
        ),
    )(x, router_W, route_idx, expert_W)


# device time: 176484 ns/iter; 1.2446x vs baseline; 1.2446x over previous
import functools

import jax
import jax.numpy as jnp
from jax import lax
from jax.experimental import pallas as pl
from jax.experimental.pallas import tpu as pltpu

N_DEV = 8
N_PLANE = 4
N_HOP = 6


def kernel(x, router_W, route_idx, expert_W):
    T, D = x.shape
    E_LOCAL, _, H = expert_W.shape
    E_HALF = E_LOCAL // 2
    N_EXP = router_W.shape[1]

    def body(x_ref, rw_ref, idx_ref, ew_ref, out_ref,
             commR_ref, commL_ref, zsrc_ref, zbuf_ref,
             sendR_sems, recvR_sems, sendL_sems, recvL_sems,
             zsend_sem, zrecv_sem, creditR_sem, creditL_sem):
        my = lax.axis_index("i")
        inplane = lax.rem(my, N_PLANE)
        pb = my - inplane
        right = pb + lax.rem(inplane + 1, N_PLANE)
        left = pb + lax.rem(inplane + N_PLANE - 1, N_PLANE)
        zp = lax.rem(my + N_PLANE, N_DEV)

        def originR(k):
            if k <= 3:
                return pb + lax.rem(inplane - k + N_PLANE, N_PLANE)
            return lax.rem(
                pb + lax.rem(inplane - (k - 3) + N_PLANE, N_PLANE) + N_PLANE,
                N_DEV)

        def originL(k):
            if k <= 3:
                return pb + lax.rem(inplane + k, N_PLANE)
            return lax.rem(
                pb + lax.rem(inplane + (k - 3), N_PLANE) + N_PLANE, N_DEV)

        barrier = pltpu.get_barrier_semaphore()
        for nbr in (left, right, zp):
            pl.semaphore_signal(barrier, inc=1, device_id=(nbr,),
                                device_id_type=pl.DeviceIdType.MESH)
        pl.semaphore_wait(barrier, 3)

        xv = x_ref[...]
        scores = jnp.dot(xv, rw_ref[...], preferred_element_type=jnp.float32)
        e0 = idx_ref[:, 0:1]
        e1 = idx_ref[:, 1:2]
        eids = lax.broadcasted_iota(jnp.int32, (T, N_EXP), 1)
        s0 = jnp.sum(jnp.where(eids == e0, scores, 0.0), axis=1)
        s1 = jnp.sum(jnp.where(eids == e1, scores, 0.0), axis=1)
        w0 = jax.nn.sigmoid(s0 - s1)
        w1 = 1.0 - w0
        xv16 = xv.astype(jnp.bfloat16)

        def contribution(block, base, n_exp):
            ge = base + lax.broadcasted_iota(jnp.int32, (1, n_exp), 1)
            coeff = (w0[:, None] * (e0 == ge).astype(jnp.float32)
                     + w1[:, None] * (e1 == ge).astype(jnp.float32)
                     ).astype(jnp.bfloat16)
            xs = (xv16[:, None, :] * coeff[:, :, None]).reshape(T, n_exp * D)
            blk = block.reshape(n_exp * D, H).astype(jnp.bfloat16)
            return jnp.dot(xs, blk, preferred_element_type=jnp.float32)

        blk16 = ew_ref[...].astype(jnp.bfloat16)
        commR_ref[0, ...] = blk16[0:E_HALF]
        commL_ref[0, ...] = blk16[E_HALF:E_LOCAL]
        zsrc_ref[...] = blk16

        zrdma = pltpu.make_async_remote_copy(
            src_ref=zsrc_ref,
            dst_ref=zbuf_ref,
            send_sem=zsend_sem,
            recv_sem=zrecv_sem,
            device_id=(zp,),
            device_id_type=pl.DeviceIdType.MESH,
        )
        zrdma.start()

        for h in range(1, N_HOP + 1):
            send_slot = (h - 1) % 2
            recv_slot = h % 2
            if h >= 2:
                pl.semaphore_wait(creditR_sem, 1)
                pl.semaphore_wait(creditL_sem, 1)
            if h == 4:
                zrdma.wait_recv()
                srcR = zbuf_ref.at[0:E_HALF]
                srcL = zbuf_ref.at[E_HALF:E_LOCAL]
            else:
                srcR = commR_ref.at[send_slot]
                srcL = commL_ref.at[send_slot]
            rdmaR = pltpu.make_async_remote_copy(
                src_ref=srcR,
                dst_ref=commR_ref.at[recv_slot],
                send_sem=sendR_sems.at[send_slot],
                recv_sem=recvR_sems.at[recv_slot],
                device_id=(right,),
                device_id_type=pl.DeviceIdType.MESH,
            )
            rdmaL = pltpu.make_async_remote_copy(
                src_ref=srcL,
                dst_ref=commL_ref.at[recv_slot],
                send_sem=sendL_sems.at[send_slot],
                recv_sem=recvL_sems.at[recv_slot],
                device_id=(left,),
                device_id_type=pl.DeviceIdType.MESH,
            )
            rdmaR.start()
            rdmaL.start()
            if h == 1:
                out_ref[...] = contribution(ew_ref[...], my * E_LOCAL,
                                            E_LOCAL)
            else:
                out_ref[...] += contribution(
                    commR_ref[send_slot], originR(h - 1) * E_LOCAL, E_HALF)
                out_ref[...] += contribution(
                    commL_ref[send_slot],
                    originL(h - 1) * E_LOCAL + E_HALF, E_HALF)
            if h == 4:
                out_ref[...] += contribution(zbuf_ref[...], zp * E_LOCAL,
                                             E_LOCAL)
            rdmaR.wait()
            rdmaL.wait()
            if h <= N_HOP - 1:
                pl.semaphore_signal(creditR_sem, inc=1, device_id=(left,),
                                    device_id_type=pl.DeviceIdType.MESH)
                pl.semaphore_signal(creditL_sem, inc=1, device_id=(right,),
                                    device_id_type=pl.DeviceIdType.MESH)
        zrdma.wait_send()
        last = N_HOP % 2
        out_ref[...] += contribution(commR_ref[last],
                                     originR(N_HOP) * E_LOCAL, E_HALF)
        out_ref[...] += contribution(commL_ref[last],
                                     originL(N_HOP) * E_LOCAL + E_HALF,
                                     E_HALF)

        @functools.partial(pl.run_scoped,
                           exit_sem=pltpu.SemaphoreType.REGULAR)
        def _(exit_sem):
            for nbr in (left, right, zp):
                pl.semaphore_signal(exit_sem, inc=1, device_id=(nbr,),
                                    device_id_type=pl.DeviceIdType.MESH)
            pl.semaphore_wait(exit_sem, 3)

    return pl.pallas_call(
        body,
        out_shape=jax.ShapeDtypeStruct((T, H), jnp.float32),
        in_specs=[pl.BlockSpec(memory_space=pltpu.VMEM)] * 4,
        out_specs=pl.BlockSpec(memory_space=pltpu.VMEM),
        scratch_shapes=[
            pltpu.VMEM((2, E_HALF, D, H), jnp.bfloat16),
            pltpu.VMEM((2, E_HALF, D, H), jnp.bfloat16),
            pltpu.VMEM((E_LOCAL, D, H), jnp.bfloat16),
            pltpu.VMEM((E_LOCAL, D, H), jnp.bfloat16),
            pltpu.SemaphoreType.DMA((2,)),
            pltpu.SemaphoreType.DMA((2,)),
            pltpu.SemaphoreType.DMA((2,)),
            pltpu.SemaphoreType.DMA((2,)),
            pltpu.SemaphoreType.DMA,
            pltpu.SemaphoreType.DMA,
            pltpu.SemaphoreType.REGULAR,
            pltpu.SemaphoreType.REGULAR,
        ],
        compiler_params=pltpu.CompilerParams(
            collective_id=0,
            vmem_limit_bytes=100 * 1024 * 1024,
        ),
    )(x, router_W, route_idx, expert_W)
